# baseline (speedup 1.0000x reference)
import numpy as np
import jax
import jax.numpy as jnp
from jax import lax
from jax.experimental import pallas as pl
from jax.experimental.pallas import tpu as pltpu

_SEQ = 2048
_D = 1024
_H = 16
_DH = 64
_BLK = 64
_NB = _SEQ // _BLK
_KMAX = 8
_NEG = -1e30
_SCALE = 1.0 / np.sqrt(_DH)


def _sparsity_tables():
    n = _NB
    rng = np.random.RandomState(0)
    mask = np.zeros((_H, n, n), dtype=bool)
    mask[:, 0, :] = True
    mask[:, -1, :] = True
    mask[:, :, 0] = True
    mask[:, :, -1] = True
    for i in range(n):
        for j in (i - 1, i, i + 1):
            if 0 <= j < n:
                mask[:, i, j] = True
    for h in range(_H):
        for i in range(1, n - 1):
            choices = rng.choice(np.arange(1, n - 1), size=3, replace=False)
            mask[h, i, choices] = True
    idx = np.zeros((_H, n, _KMAX), dtype=np.int32)
    pen = np.full((_H, n, _KMAX), np.float32(_NEG), dtype=np.float32)
    for h in range(_H):
        for i in range(1, n - 1):
            cols = np.nonzero(mask[h, i])[0]
            assert len(cols) <= _KMAX
            idx[h, i, : len(cols)] = cols.astype(np.int32)
            pen[h, i, : len(cols)] = 0.0
    return idx, pen


_IDX_TAB, _PEN_TAB = _sparsity_tables()


def _attn_kernel(idx_ref, pen_ref, x_ref, wq_ref, wk_ref, wv_ref, o_ref,
                 q_s, k_s, v_s):
    hp = pl.program_id(0)
    i = pl.program_id(1)

    @pl.when(jnp.logical_and(hp == 0, i == 0))
    def _project():
        x = x_ref[...]
        q_s[...] = jnp.dot(
            x, wq_ref[...], preferred_element_type=jnp.float32
        ).astype(jnp.bfloat16)
        k_s[...] = jnp.dot(
            x, wk_ref[...], preferred_element_type=jnp.float32
        ).astype(jnp.bfloat16)
        v_s[...] = jnp.dot(
            x, wv_ref[...], preferred_element_type=jnp.float32
        ).astype(jnp.bfloat16)

    hc = hp * (2 * _DH)
    q_pair = q_s[pl.ds(i * _BLK, _BLK), pl.ds(hc, 2 * _DH)] * jnp.bfloat16(_SCALE)
    lane = lax.broadcasted_iota(jnp.int32, (_BLK, 2 * _DH), 1)
    m0 = (lane < _DH).astype(jnp.float32)
    m1 = 1.0 - m0
    m0b = m0.astype(jnp.bfloat16)
    m1b = m1.astype(jnp.bfloat16)
    qh = (q_pair * m0b, q_pair * m1b)

    is_global = jnp.logical_or(i == 0, i == _NB - 1)

    @pl.when(is_global)
    def _dense_row():
        kh = k_s[:, pl.ds(hc, 2 * _DH)]
        vh = v_s[:, pl.ds(hc, 2 * _DH)]
        ctx = []
        for d in range(2):
            s = lax.dot_general(qh[d], kh, (((1,), (1,)), ((), ())),
                                preferred_element_type=jnp.float32)
            m = jnp.max(s, axis=1, keepdims=True)
            e = jnp.exp(s - m)
            inv = 1.0 / jnp.sum(e, axis=1, keepdims=True)
            ctx.append(jnp.dot(e.astype(jnp.bfloat16), vh,
                               preferred_element_type=jnp.float32) * inv)
        o_ref[...] = ctx[0] * m0 + ctx[1] * m1

    @pl.when(jnp.logical_not(is_global))
    def _sparse_row():
        ctx = []
        for d in range(2):
            h = hp * 2 + d
            ss = []
            for kk in range(_KMAX):
                j = idx_ref[h, i, kk]
                kb = k_s[pl.ds(j * _BLK, _BLK), pl.ds(hc, 2 * _DH)]
                s = lax.dot_general(qh[d], kb, (((1,), (1,)), ((), ())),
                                    preferred_element_type=jnp.float32)
                ss.append(s + pen_ref[h, i, kk])
            m = ss[0].max(axis=1, keepdims=True)
            for s in ss[1:]:
                m = jnp.maximum(m, s.max(axis=1, keepdims=True))
            es = [jnp.exp(s - m) for s in ss]
            denom = es[0].sum(axis=1, keepdims=True)
            for e in es[1:]:
                denom = denom + e.sum(axis=1, keepdims=True)
            acc = jnp.zeros((_BLK, 2 * _DH), dtype=jnp.float32)
            for kk in range(_KMAX):
                j = idx_ref[h, i, kk]
                vb = v_s[pl.ds(j * _BLK, _BLK), pl.ds(hc, 2 * _DH)]
                acc = acc + jnp.dot(es[kk].astype(jnp.bfloat16), vb,
                                    preferred_element_type=jnp.float32)
            ctx.append(acc * (1.0 / denom))
        o_ref[...] = ctx[0] * m0 + ctx[1] * m1


def _run(x, Wq, Wk, Wv, interpret=False):
    grid_spec = pltpu.PrefetchScalarGridSpec(
        num_scalar_prefetch=2,
        grid=(_H // 2, _NB),
        in_specs=[
            pl.BlockSpec((_SEQ, _D), lambda hp, i, *_: (0, 0)),
            pl.BlockSpec((_D, _D), lambda hp, i, *_: (0, 0)),
            pl.BlockSpec((_D, _D), lambda hp, i, *_: (0, 0)),
            pl.BlockSpec((_D, _D), lambda hp, i, *_: (0, 0)),
        ],
        out_specs=pl.BlockSpec((_BLK, 2 * _DH), lambda hp, i, *_: (i, hp)),
        scratch_shapes=[
            pltpu.VMEM((_SEQ, _D), jnp.bfloat16),
            pltpu.VMEM((_SEQ, _D), jnp.bfloat16),
            pltpu.VMEM((_SEQ, _D), jnp.bfloat16),
        ],
    )
    return pl.pallas_call(
        _attn_kernel,
        grid_spec=grid_spec,
        out_shape=jax.ShapeDtypeStruct((_SEQ, _D), jnp.float32),
        interpret=interpret,
    )(jnp.asarray(_IDX_TAB), jnp.asarray(_PEN_TAB), x, Wq, Wk, Wv)


def kernel(hidden_states, Wq, Wk, Wv):
    x = hidden_states[0].astype(jnp.bfloat16)
    return _run(x, Wq.astype(jnp.bfloat16), Wk.astype(jnp.bfloat16),
                Wv.astype(jnp.bfloat16))[None]


# speedup vs baseline: 1.3505x; 1.3505x over previous
import numpy as np
import jax
import jax.numpy as jnp
from jax import lax
from jax.experimental import pallas as pl
from jax.experimental.pallas import tpu as pltpu

_SEQ = 2048
_D = 1024
_H = 16
_DH = 64
_BLK = 64
_NB = _SEQ // _BLK
_KMAX = 8
_NEG = -1e30
_SCALE = 1.0 / np.sqrt(_DH)
_NHQ = 4
_W = _NHQ * _DH


def _sparsity_tables():
    n = _NB
    rng = np.random.RandomState(0)
    mask = np.zeros((_H, n, n), dtype=bool)
    mask[:, 0, :] = True
    mask[:, -1, :] = True
    mask[:, :, 0] = True
    mask[:, :, -1] = True
    for i in range(n):
        for j in (i - 1, i, i + 1):
            if 0 <= j < n:
                mask[:, i, j] = True
    for h in range(_H):
        for i in range(1, n - 1):
            choices = rng.choice(np.arange(1, n - 1), size=3, replace=False)
            mask[h, i, choices] = True
    idx = np.zeros((_H, n, _KMAX), dtype=np.int32)
    pen = np.full((_H, n, _KMAX), np.float32(_NEG), dtype=np.float32)
    for h in range(_H):
        for i in range(1, n - 1):
            cols = np.nonzero(mask[h, i])[0]
            assert len(cols) <= _KMAX
            idx[h, i, : len(cols)] = cols.astype(np.int32)
            pen[h, i, : len(cols)] = 0.0
    return idx, pen


_IDX_TAB, _PEN_TAB = _sparsity_tables()


def _attn_kernel(idx_ref, pen_ref, x_ref, wq_ref, wk_ref, wv_ref, o_ref,
                 q_s, k_s, v_s):
    hq = pl.program_id(0)
    i = pl.program_id(1)

    @pl.when(jnp.logical_and(hq == 0, i == 0))
    def _project():
        x = x_ref[...]
        q_s[...] = jnp.dot(
            x, wq_ref[...], preferred_element_type=jnp.float32
        ).astype(jnp.bfloat16)
        k_s[...] = jnp.dot(
            x, wk_ref[...], preferred_element_type=jnp.float32
        ).astype(jnp.bfloat16)
        v_s[...] = jnp.dot(
            x, wv_ref[...], preferred_element_type=jnp.float32
        ).astype(jnp.bfloat16)

    hc = hq * _W
    q_quad = q_s[pl.ds(i * _BLK, _BLK), pl.ds(hc, _W)] * jnp.bfloat16(_SCALE)
    lane = lax.broadcasted_iota(jnp.int32, (_BLK, _W), 1)
    mf = [( (lane >= d * _DH) & (lane < (d + 1) * _DH) ).astype(jnp.float32)
          for d in range(_NHQ)]
    mb = [m.astype(jnp.bfloat16) for m in mf]
    qh = [q_quad * mb[d] for d in range(_NHQ)]

    is_global = jnp.logical_or(i == 0, i == _NB - 1)

    @pl.when(is_global)
    def _dense_row():
        kh = k_s[:, pl.ds(hc, _W)]
        vh = v_s[:, pl.ds(hc, _W)]
        out = jnp.zeros((_BLK, _W), dtype=jnp.float32)
        for d in range(_NHQ):
            s = lax.dot_general(qh[d], kh, (((1,), (1,)), ((), ())),
                                preferred_element_type=jnp.float32)
            m = jnp.max(s, axis=1, keepdims=True)
            e = jnp.exp(s - m)
            inv = 1.0 / jnp.sum(e, axis=1, keepdims=True)
            ctx = jnp.dot(e.astype(jnp.bfloat16), vh,
                          preferred_element_type=jnp.float32) * inv
            out = out + ctx * mf[d]
        o_ref[...] = out

    @pl.when(jnp.logical_not(is_global))
    def _sparse_row():
        out = jnp.zeros((_BLK, _W), dtype=jnp.float32)
        for d in range(_NHQ):
            h = hq * _NHQ + d
            ss = []
            for kk in range(_KMAX):
                j = idx_ref[h, i, kk]
                kb = k_s[pl.ds(j * _BLK, _BLK), pl.ds(hc, _W)]
                s = lax.dot_general(qh[d], kb, (((1,), (1,)), ((), ())),
                                    preferred_element_type=jnp.float32)
                ss.append(s + pen_ref[h, i, kk])
            m = ss[0].max(axis=1, keepdims=True)
            for s in ss[1:]:
                m = jnp.maximum(m, s.max(axis=1, keepdims=True))
            es = [jnp.exp(s - m) for s in ss]
            denom = es[0].sum(axis=1, keepdims=True)
            for e in es[1:]:
                denom = denom + e.sum(axis=1, keepdims=True)
            acc = jnp.zeros((_BLK, _W), dtype=jnp.float32)
            for kk in range(_KMAX):
                j = idx_ref[h, i, kk]
                vb = v_s[pl.ds(j * _BLK, _BLK), pl.ds(hc, _W)]
                acc = acc + jnp.dot(es[kk].astype(jnp.bfloat16), vb,
                                    preferred_element_type=jnp.float32)
            out = out + acc * (1.0 / denom) * mf[d]
        o_ref[...] = out


def _run(x, Wq, Wk, Wv, interpret=False):
    grid_spec = pltpu.PrefetchScalarGridSpec(
        num_scalar_prefetch=2,
        grid=(_H // _NHQ, _NB),
        in_specs=[
            pl.BlockSpec((_SEQ, _D), lambda hq, i, *_: (0, 0)),
            pl.BlockSpec((_D, _D), lambda hq, i, *_: (0, 0)),
            pl.BlockSpec((_D, _D), lambda hq, i, *_: (0, 0)),
            pl.BlockSpec((_D, _D), lambda hq, i, *_: (0, 0)),
        ],
        out_specs=pl.BlockSpec((_BLK, _W), lambda hq, i, *_: (i, hq)),
        scratch_shapes=[
            pltpu.VMEM((_SEQ, _D), jnp.bfloat16),
            pltpu.VMEM((_SEQ, _D), jnp.bfloat16),
            pltpu.VMEM((_SEQ, _D), jnp.bfloat16),
        ],
    )
    return pl.pallas_call(
        _attn_kernel,
        grid_spec=grid_spec,
        out_shape=jax.ShapeDtypeStruct((_SEQ, _D), jnp.float32),
        interpret=interpret,
    )(jnp.asarray(_IDX_TAB), jnp.asarray(_PEN_TAB), x, Wq, Wk, Wv)


def kernel(hidden_states, Wq, Wk, Wv):
    x = hidden_states[0].astype(jnp.bfloat16)
    return _run(x, Wq.astype(jnp.bfloat16), Wk.astype(jnp.bfloat16),
                Wv.astype(jnp.bfloat16))[None]


# speedup vs baseline: 1.6966x; 1.2563x over previous
import numpy as np
import jax
import jax.numpy as jnp
from jax import lax
from jax.experimental import pallas as pl
from jax.experimental.pallas import tpu as pltpu

_SEQ = 2048
_D = 1024
_H = 16
_DH = 64
_BLK = 64
_NB = _SEQ // _BLK
_KMAX = 8
_NEG = -1e30
_SCALE = 1.0 / np.sqrt(_DH)
_NHQ = 4
_W = _NHQ * _DH
_NR = 8
_NG = _NB // _NR


def _sparsity_tables():
    n = _NB
    rng = np.random.RandomState(0)
    mask = np.zeros((_H, n, n), dtype=bool)
    mask[:, 0, :] = True
    mask[:, -1, :] = True
    mask[:, :, 0] = True
    mask[:, :, -1] = True
    for i in range(n):
        for j in (i - 1, i, i + 1):
            if 0 <= j < n:
                mask[:, i, j] = True
    for h in range(_H):
        for i in range(1, n - 1):
            choices = rng.choice(np.arange(1, n - 1), size=3, replace=False)
            mask[h, i, choices] = True
    idx = np.zeros((_H, n, _KMAX), dtype=np.int32)
    pen = np.full((_H, n, _KMAX), np.float32(_NEG), dtype=np.float32)
    for h in range(_H):
        for i in range(1, n - 1):
            cols = np.nonzero(mask[h, i])[0]
            assert len(cols) <= _KMAX
            idx[h, i, : len(cols)] = cols.astype(np.int32)
            pen[h, i, : len(cols)] = 0.0
    return idx, pen


_IDX_TAB, _PEN_TAB = _sparsity_tables()


def _attn_kernel(idx_ref, pen_ref, x_ref, wq_ref, wk_ref, wv_ref, o_ref,
                 q_s, k_s, v_s):
    hq = pl.program_id(0)
    g = pl.program_id(1)

    @pl.when(jnp.logical_and(hq == 0, g == 0))
    def _project():
        x = x_ref[...]
        q_s[...] = jnp.dot(
            x, wq_ref[...], preferred_element_type=jnp.float32
        ).astype(jnp.bfloat16)
        k_s[...] = jnp.dot(
            x, wk_ref[...], preferred_element_type=jnp.float32
        ).astype(jnp.bfloat16)
        v_s[...] = jnp.dot(
            x, wv_ref[...], preferred_element_type=jnp.float32
        ).astype(jnp.bfloat16)

    hc = hq * _W
    lane = lax.broadcasted_iota(jnp.int32, (_BLK, _W), 1)
    mf = [((lane >= d * _DH) & (lane < (d + 1) * _DH)).astype(jnp.float32)
          for d in range(_NHQ)]
    mb = [m.astype(jnp.bfloat16) for m in mf]

    def dense_unit(qh, d):
        kh = k_s[:, pl.ds(hc, _W)]
        vh = v_s[:, pl.ds(hc, _W)]
        s = lax.dot_general(qh, kh, (((1,), (1,)), ((), ())),
                            preferred_element_type=jnp.float32)
        m = jnp.max(s, axis=1, keepdims=True)
        e = jnp.exp(s - m)
        inv = 1.0 / jnp.sum(e, axis=1, keepdims=True)
        return jnp.dot(e.astype(jnp.bfloat16), vh,
                       preferred_element_type=jnp.float32) * inv

    def sparse_unit(qh, h, i):
        ss = []
        for kk in range(_KMAX):
            j = idx_ref[h, i, kk]
            kb = k_s[pl.ds(j * _BLK, _BLK), pl.ds(hc, _W)]
            s = lax.dot_general(qh, kb, (((1,), (1,)), ((), ())),
                                preferred_element_type=jnp.float32)
            ss.append(s + pen_ref[h, i, kk])
        m = ss[0].max(axis=1, keepdims=True)
        for s in ss[1:]:
            m = jnp.maximum(m, s.max(axis=1, keepdims=True))
        es = [jnp.exp(s - m) for s in ss]
        denom = es[0].sum(axis=1, keepdims=True)
        for e in es[1:]:
            denom = denom + e.sum(axis=1, keepdims=True)
        acc = jnp.zeros((_BLK, _W), dtype=jnp.float32)
        for kk in range(_KMAX):
            j = idx_ref[h, i, kk]
            vb = v_s[pl.ds(j * _BLK, _BLK), pl.ds(hc, _W)]
            acc = acc + jnp.dot(es[kk].astype(jnp.bfloat16), vb,
                                preferred_element_type=jnp.float32)
        return acc * (1.0 / denom)

    for ii in range(_NR):
        i = g * _NR + ii
        q_quad = q_s[pl.ds(i * _BLK, _BLK), pl.ds(hc, _W)] * jnp.bfloat16(_SCALE)
        qhs = [q_quad * mb[d] for d in range(_NHQ)]

        if ii == 0:
            @pl.when(g == 0)
            def _g0():
                out = jnp.zeros((_BLK, _W), dtype=jnp.float32)
                for d in range(_NHQ):
                    out = out + dense_unit(qhs[d], d) * mf[d]
                o_ref[ii * _BLK:(ii + 1) * _BLK, :] = out

            @pl.when(g != 0)
            def _s0():
                out = jnp.zeros((_BLK, _W), dtype=jnp.float32)
                for d in range(_NHQ):
                    out = out + sparse_unit(qhs[d], hq * _NHQ + d, i) * mf[d]
                o_ref[ii * _BLK:(ii + 1) * _BLK, :] = out
        elif ii == _NR - 1:
            @pl.when(g == _NG - 1)
            def _g3():
                out = jnp.zeros((_BLK, _W), dtype=jnp.float32)
                for d in range(_NHQ):
                    out = out + dense_unit(qhs[d], d) * mf[d]
                o_ref[ii * _BLK:(ii + 1) * _BLK, :] = out

            @pl.when(g != _NG - 1)
            def _s7():
                out = jnp.zeros((_BLK, _W), dtype=jnp.float32)
                for d in range(_NHQ):
                    out = out + sparse_unit(qhs[d], hq * _NHQ + d, i) * mf[d]
                o_ref[ii * _BLK:(ii + 1) * _BLK, :] = out
        else:
            out = jnp.zeros((_BLK, _W), dtype=jnp.float32)
            for d in range(_NHQ):
                out = out + sparse_unit(qhs[d], hq * _NHQ + d, i) * mf[d]
            o_ref[ii * _BLK:(ii + 1) * _BLK, :] = out


def _run(x, Wq, Wk, Wv, interpret=False):
    grid_spec = pltpu.PrefetchScalarGridSpec(
        num_scalar_prefetch=2,
        grid=(_H // _NHQ, _NG),
        in_specs=[
            pl.BlockSpec((_SEQ, _D), lambda hq, g, *_: (0, 0)),
            pl.BlockSpec((_D, _D), lambda hq, g, *_: (0, 0)),
            pl.BlockSpec((_D, _D), lambda hq, g, *_: (0, 0)),
            pl.BlockSpec((_D, _D), lambda hq, g, *_: (0, 0)),
        ],
        out_specs=pl.BlockSpec((_NR * _BLK, _W), lambda hq, g, *_: (g, hq)),
        scratch_shapes=[
            pltpu.VMEM((_SEQ, _D), jnp.bfloat16),
            pltpu.VMEM((_SEQ, _D), jnp.bfloat16),
            pltpu.VMEM((_SEQ, _D), jnp.bfloat16),
        ],
    )
    return pl.pallas_call(
        _attn_kernel,
        grid_spec=grid_spec,
        out_shape=jax.ShapeDtypeStruct((_SEQ, _D), jnp.float32),
        interpret=interpret,
    )(jnp.asarray(_IDX_TAB), jnp.asarray(_PEN_TAB), x, Wq, Wk, Wv)


def kernel(hidden_states, Wq, Wk, Wv):
    x = hidden_states[0].astype(jnp.bfloat16)
    return _run(x, Wq.astype(jnp.bfloat16), Wk.astype(jnp.bfloat16),
                Wv.astype(jnp.bfloat16))[None]
